# baseline (speedup 1.0000x reference)
import functools

import jax
import jax.numpy as jnp
from jax import lax
from jax.experimental import pallas as pl
from jax.experimental.pallas import tpu as pltpu
from jax.experimental.pallas import tpu_sc as plsc

N = 10000
D = 256
F = 64
E = 160000

NC = 2
NS = 16
NW = NC * NS

CW = 128
ROWS = 1280
EPAD = ROWS * CW - E
RPW = ROWS // NW
EPW = RPW * CW
SBR = 8
NSB = RPW // SBR

NJ = N
APS = 640
NPAD = NS * APS
ZR = 160

_HIGH = jax.lax.Precision.HIGHEST


def _mesh():
    return plsc.VectorSubcoreMesh(
        core_axis_name="c", subcore_axis_name="s", num_cores=NC,
        num_subcores=NS)


def _sc_degree_body(dst_hbm, out_hbm, didx, hist, tmp, accb, hist_all):
    c = lax.axis_index("c")
    s = lax.axis_index("s")
    wid = s * NC + c

    def zh(i, carry):
        hist[pl.ds(i * 16, 16)] = jnp.zeros((16,), jnp.float32)
        return carry
    lax.fori_loop(0, NPAD // 16, zh, 0)

    pltpu.sync_copy(dst_hbm.at[pl.ds(wid * EPW, EPW)], didx)
    iota = lax.iota(jnp.int32, 16)
    onesf = jnp.ones((16,), jnp.float32)

    def hbody(k, carry):
        idx = didx[pl.ds(k * 16, 16)]
        for l in range(16):
            plsc.addupdate_scatter(hist, [idx], onesf, mask=iota == l)
        return carry
    lax.fori_loop(0, EPW // 16, hbody, 0)

    pltpu.sync_copy(hist, hist_all.at[pl.ds(s * NPAD, NPAD)])
    plsc.subcore_barrier()
    pltpu.sync_copy(hist_all.at[pl.ds(s * APS, APS)], accb)
    for t in range(1, NS):
        pltpu.sync_copy(hist_all.at[pl.ds(t * NPAD + s * APS, APS)], tmp)

        def radd(k, carry):
            sl = pl.ds(k * 16, 16)
            accb[sl] = accb[sl] + tmp[sl]
            return carry
        lax.fori_loop(0, APS // 16, radd, 0)
    pltpu.sync_copy(accb, out_hbm.at[pl.ds(c * NPAD + s * APS, APS)])


@functools.lru_cache(maxsize=None)
def _sc_degree():
    return pl.kernel(
        _sc_degree_body,
        out_type=jax.ShapeDtypeStruct((NC * NPAD,), jnp.float32),
        mesh=_mesh(),
        scratch_types=[
            pltpu.VMEM((EPW,), jnp.int32),
            pltpu.VMEM((NPAD,), jnp.float32),
            pltpu.VMEM((APS,), jnp.float32),
            pltpu.VMEM((APS,), jnp.float32),
            pltpu.VMEM_SHARED((NS * NPAD,), jnp.float32),
        ],
        compiler_params=pltpu.CompilerParams(
            use_tc_tiling_on_sc=False, needs_layout_passes=False),
    )


NB = 4
KPF = 2
RA = ROWS // NS


def _sc_prop_body(x_hbm, src_hbm, dst_hbm, out_hbm, sidx, didx, rows, zrow,
                  acc, gsem, ssem):
    c = lax.axis_index("c")
    s = lax.axis_index("s")

    def _prop_core0():
        def zb(i, carry):
            for j in range(F // 16):
                zrow[i, pl.ds(j * 16, 16)] = jnp.zeros((16,), jnp.float32)
            return carry
        lax.fori_loop(0, ZR, zb, 0)
        for r in range(APS // ZR):
            pltpu.sync_copy(zrow, acc.at[pl.ds(s * APS + r * ZR, ZR)])
        plsc.subcore_barrier()

        def edge_loop(base, nch):
            pltpu.sync_copy(src_hbm.at[pl.ds(base, nch)],
                            sidx.at[pl.ds(0, nch)])
            pltpu.sync_copy(dst_hbm.at[pl.ds(base, nch)],
                            didx.at[pl.ds(0, nch)])
            dg = [None] * nch
            dsc = [None] * nch
            for j in range(KPF):
                dg[j] = pltpu.async_copy(x_hbm.at[sidx.at[j]],
                                         rows.at[j % NB], gsem.at[j % NB])
            for j in range(nch):
                jn = j + KPF
                if jn < nch:
                    b2 = jn % NB
                    if jn >= NB:
                        dsc[jn - NB].wait()
                    dg[jn] = pltpu.async_copy(x_hbm.at[sidx.at[jn]],
                                              rows.at[b2], gsem.at[b2])
                b = j % NB
                dg[j].wait()
                dsc[j] = pltpu.async_copy(rows.at[b], acc.at[didx.at[j]],
                                          ssem.at[b], add=True)
            for j in range(max(nch - NB, 0), nch):
                dsc[j].wait()

        edge_loop(s * RA, RA)
        plsc.subcore_barrier()
        pltpu.sync_copy(acc.at[pl.ds(s * APS, APS)],
                        out_hbm.at[pl.ds(s * APS, APS)])

    @pl.when(c == 0)
    def _():
        _prop_core0()


@functools.lru_cache(maxsize=None)
def _sc_prop():
    return pl.kernel(
        _sc_prop_body,
        out_type=jax.ShapeDtypeStruct((NPAD, F), jnp.float32),
        mesh=_mesh(),
        scratch_types=[
            pltpu.VMEM((RA, CW), jnp.int32),
            pltpu.VMEM((RA, CW), jnp.int32),
            pltpu.VMEM((NB, CW, F), jnp.float32),
            pltpu.VMEM((ZR, F), jnp.float32),
            pltpu.VMEM_SHARED((NPAD, F), jnp.float32),
            pltpu.SemaphoreType.DMA((NB,)),
            pltpu.SemaphoreType.DMA((NB,)),
        ],
        compiler_params=pltpu.CompilerParams(use_tc_tiling_on_sc=False),
    )


def _fold_body(w0, w1, w2, fc1, fc2, bb0, bb1, bb2, m_ref, c_ref):
    fc = jnp.dot(fc1[...], fc2[...], precision=_HIGH,
                 preferred_element_type=jnp.float32)
    m_ref[:, 0:F] = jnp.dot(w0[...], fc[0:D], precision=_HIGH,
                            preferred_element_type=jnp.float32)
    m_ref[:, F:2 * F] = jnp.dot(w1[...], fc[D:2 * D], precision=_HIGH,
                                preferred_element_type=jnp.float32)
    m_ref[:, 2 * F:3 * F] = jnp.dot(w2[...], fc[2 * D:3 * D], precision=_HIGH,
                                    preferred_element_type=jnp.float32)
    c_ref[...] = (
        jnp.dot(bb0[...], fc[0:D], precision=_HIGH,
                preferred_element_type=jnp.float32)
        + jnp.dot(bb1[...], fc[D:2 * D], precision=_HIGH,
                  preferred_element_type=jnp.float32)
        + jnp.dot(bb2[...], fc[2 * D:3 * D], precision=_HIGH,
                  preferred_element_type=jnp.float32))


def _tc_fold(W0, W1, W2, FC1, FC2, b0, b1, b2):
    return pl.pallas_call(
        _fold_body,
        out_shape=[jax.ShapeDtypeStruct((D, 3 * F), jnp.float32),
                   jax.ShapeDtypeStruct((1, F), jnp.float32)],
    )(W0, W1, W2, FC1, FC2, b0, b1, b2)


BN = 1000


def _main_body(feat_b, m_b, d0_b, d1_b, y0, y1n, x2, nrm):
    nv = lax.rsqrt(jnp.maximum(d0_b[...] + d1_b[...], 1.0))
    y = jnp.dot(feat_b[...], m_b[...], precision=_HIGH,
                preferred_element_type=jnp.float32)
    y0[...] = y[:, 0:F]
    y1n[...] = y[:, F:2 * F] * nv
    x2[...] = y[:, 2 * F:3 * F] * nv
    nrm[...] = nv


def _tc_main(feat, M, d0, d1):
    bs = pl.BlockSpec((BN, F), lambda i: (i, 0))
    b1 = pl.BlockSpec((BN, 1), lambda i: (i, 0))
    return pl.pallas_call(
        _main_body,
        grid=(N // BN,),
        in_specs=[
            pl.BlockSpec((BN, D), lambda i: (i, 0)),
            pl.BlockSpec((D, 3 * F), lambda i: (0, 0)),
            b1,
            b1,
        ],
        out_specs=[bs, bs, bs, b1],
        out_shape=[jax.ShapeDtypeStruct((N, F), jnp.float32),
                   jax.ShapeDtypeStruct((N, F), jnp.float32),
                   jax.ShapeDtypeStruct((N, F), jnp.float32),
                   jax.ShapeDtypeStruct((N, 1), jnp.float32)],
    )(feat, M, d0, d1)


def _combine_body(y1n_b, nrm_b, r_b, xu):
    nv = nrm_b[...]
    xu[...] = y1n_b[...] + (nv * nv) * r_b[...]


def _tc_combine(Y1n, nrm, R):
    bs = pl.BlockSpec((BN, F), lambda i: (i, 0))
    return pl.pallas_call(
        _combine_body,
        grid=(N // BN,),
        in_specs=[bs, pl.BlockSpec((BN, 1), lambda i: (i, 0)), bs],
        out_specs=bs,
        out_shape=jax.ShapeDtypeStruct((N, F), jnp.float32),
    )(Y1n, nrm, R)


def _final_body(y0_b, nrm_b, q_b, c_b, out):
    out[...] = y0_b[...] + nrm_b[...] * q_b[...] + c_b[...]


def _tc_final(Y0, nrm, Q, cvec):
    bs = pl.BlockSpec((BN, F), lambda i: (i, 0))
    return pl.pallas_call(
        _final_body,
        grid=(N // BN,),
        in_specs=[bs, pl.BlockSpec((BN, 1), lambda i: (i, 0)), bs,
                  pl.BlockSpec((1, F), lambda i: (0, 0))],
        out_specs=bs,
        out_shape=jax.ShapeDtypeStruct((N, F), jnp.float32),
    )(Y0, nrm, Q, cvec)


def kernel(feat, edge_index, W0, b0, W1, b1, W2, b2, FC1, FC2):
    src = edge_index[0]
    dst = edge_index[1]
    srcp = jnp.concatenate(
        [src, jnp.zeros((EPAD,), jnp.int32)]).reshape(ROWS, CW)
    dst_flat = jnp.concatenate([dst, jnp.full((EPAD,), NJ, jnp.int32)])
    dstp = dst_flat.reshape(ROWS, CW)

    degP = _sc_degree()(dst_flat).reshape(NC, NPAD)
    d0 = degP[0, :N].reshape(N, 1)
    d1 = degP[1, :N].reshape(N, 1)

    M, cvec = _tc_fold(W0, W1, W2, FC1, FC2,
                       b0.reshape(1, D), b1.reshape(1, D), b2.reshape(1, D))
    Y0, Y1n, X2, nrm = _tc_main(feat, M, d0, d1)

    R = _sc_prop()(X2, srcp, dstp)
    Xu = _tc_combine(Y1n, nrm, R[:N])
    Q = _sc_prop()(Xu, srcp, dstp)
    out = _tc_final(Y0, nrm, Q[:N], cvec)
    return out


# speedup vs baseline: 1.7925x; 1.7925x over previous
import functools

import jax
import jax.numpy as jnp
from jax import lax
from jax.experimental import pallas as pl
from jax.experimental.pallas import tpu as pltpu
from jax.experimental.pallas import tpu_sc as plsc

N = 10000
D = 256
F = 64
E = 160000

NC = 2
NS = 16
NW = NC * NS

CW = 128
ROWS = E // CW
EC0 = 5120
EC1 = (E - NS * EC0) // NS
RH = 79
RL = 78

APS = 640
NPAD = NS * APS
ZR = 160

_HIGH = jax.lax.Precision.HIGHEST


def _mesh():
    return plsc.VectorSubcoreMesh(
        core_axis_name="c", subcore_axis_name="s", num_cores=NC,
        num_subcores=NS)


def _sc_degree_body(dst_hbm, out_hbm, didx, hist, tmp, accb, hist_all):
    c = lax.axis_index("c")
    s = lax.axis_index("s")

    def zh(i, carry):
        hist[pl.ds(i * 16, 16)] = jnp.zeros((16,), jnp.float32)
        return carry
    lax.fori_loop(0, NPAD // 16, zh, 0)

    iota = lax.iota(jnp.int32, 16)
    onesf = jnp.ones((16,), jnp.float32)

    def hist_edges(base, cnt):
        pltpu.sync_copy(dst_hbm.at[pl.ds(base, cnt)], didx.at[pl.ds(0, cnt)])

        def hbody(k, carry):
            idx = didx[pl.ds(k * 16, 16)]
            for l in range(16):
                plsc.addupdate_scatter(hist, [idx], onesf, mask=iota == l)
            return carry
        lax.fori_loop(0, cnt // 16, hbody, 0)

    @pl.when(c == 0)
    def _():
        hist_edges(s * EC0, EC0)

    @pl.when(c == 1)
    def _():
        hist_edges(NS * EC0 + s * EC1, EC1)

    pltpu.sync_copy(hist, hist_all.at[pl.ds(s * NPAD, NPAD)])
    plsc.subcore_barrier()
    pltpu.sync_copy(hist_all.at[pl.ds(s * APS, APS)], accb)
    for t in range(1, NS):
        pltpu.sync_copy(hist_all.at[pl.ds(t * NPAD + s * APS, APS)], tmp)

        def radd(k, carry):
            sl = pl.ds(k * 16, 16)
            accb[sl] = accb[sl] + tmp[sl]
            return carry
        lax.fori_loop(0, APS // 16, radd, 0)
    pltpu.sync_copy(accb, out_hbm.at[pl.ds(c * NPAD + s * APS, APS)])


@functools.lru_cache(maxsize=None)
def _sc_degree():
    return pl.kernel(
        _sc_degree_body,
        out_type=jax.ShapeDtypeStruct((NC * NPAD,), jnp.float32),
        mesh=_mesh(),
        scratch_types=[
            pltpu.VMEM((EC0,), jnp.int32),
            pltpu.VMEM((NPAD,), jnp.float32),
            pltpu.VMEM((APS,), jnp.float32),
            pltpu.VMEM((APS,), jnp.float32),
            pltpu.VMEM_SHARED((NS * NPAD,), jnp.float32),
        ],
        compiler_params=pltpu.CompilerParams(
            use_tc_tiling_on_sc=False, needs_layout_passes=False),
    )


NB = 4
KPF = 2


def _sc_prop_body(x_hbm, src_hbm, dst_hbm, out_hbm, sidx, didx, rows, zrow,
                  acc, gsem, ssem):
    c = lax.axis_index("c")
    s = lax.axis_index("s")

    def _prop_core0():
        def zb(i, carry):
            for j in range(F // 16):
                zrow[i, pl.ds(j * 16, 16)] = jnp.zeros((16,), jnp.float32)
            return carry
        lax.fori_loop(0, ZR, zb, 0)
        for r in range(APS // ZR):
            pltpu.sync_copy(zrow, acc.at[pl.ds(s * APS + r * ZR, ZR)])
        plsc.subcore_barrier()

        def edge_loop(base, nch):
            pltpu.sync_copy(src_hbm.at[pl.ds(base, nch)],
                            sidx.at[pl.ds(0, nch)])
            pltpu.sync_copy(dst_hbm.at[pl.ds(base, nch)],
                            didx.at[pl.ds(0, nch)])
            dg = [None] * nch
            dsc = [None] * nch
            for j in range(KPF):
                dg[j] = pltpu.async_copy(x_hbm.at[sidx.at[j]],
                                         rows.at[j % NB], gsem.at[j % NB])
            for j in range(nch):
                jn = j + KPF
                if jn < nch:
                    b2 = jn % NB
                    if jn >= NB:
                        dsc[jn - NB].wait()
                    dg[jn] = pltpu.async_copy(x_hbm.at[sidx.at[jn]],
                                              rows.at[b2], gsem.at[b2])
                b = j % NB
                dg[j].wait()
                dsc[j] = pltpu.async_copy(rows.at[b], acc.at[didx.at[j]],
                                          ssem.at[b], add=True)
            for j in range(max(nch - NB, 0), nch):
                dsc[j].wait()

        @pl.when(s < 2)
        def _():
            edge_loop(s * RH, RH)

        @pl.when(s >= 2)
        def _():
            edge_loop(2 + s * RL, RL)

        plsc.subcore_barrier()
        pltpu.sync_copy(acc.at[pl.ds(s * APS, APS)],
                        out_hbm.at[pl.ds(s * APS, APS)])

    @pl.when(c == 0)
    def _():
        _prop_core0()


@functools.lru_cache(maxsize=None)
def _sc_prop():
    return pl.kernel(
        _sc_prop_body,
        out_type=jax.ShapeDtypeStruct((NPAD, F), jnp.float32),
        mesh=_mesh(),
        scratch_types=[
            pltpu.VMEM((RH, CW), jnp.int32),
            pltpu.VMEM((RH, CW), jnp.int32),
            pltpu.VMEM((NB, CW, F), jnp.float32),
            pltpu.VMEM((ZR, F), jnp.float32),
            pltpu.VMEM_SHARED((NPAD, F), jnp.float32),
            pltpu.SemaphoreType.DMA((NB,)),
            pltpu.SemaphoreType.DMA((NB,)),
        ],
        compiler_params=pltpu.CompilerParams(use_tc_tiling_on_sc=False),
    )


def _fold_body(w0, w1, w2, fc1, fc2, bb0, bb1, bb2, m_ref, c_ref):
    fc = jnp.dot(fc1[...], fc2[...], precision=_HIGH,
                 preferred_element_type=jnp.float32)
    m_ref[:, 0:F] = jnp.dot(w0[...], fc[0:D], precision=_HIGH,
                            preferred_element_type=jnp.float32)
    m_ref[:, F:2 * F] = jnp.dot(w1[...], fc[D:2 * D], precision=_HIGH,
                                preferred_element_type=jnp.float32)
    m_ref[:, 2 * F:3 * F] = jnp.dot(w2[...], fc[2 * D:3 * D], precision=_HIGH,
                                    preferred_element_type=jnp.float32)
    c_ref[...] = (
        jnp.dot(bb0[...], fc[0:D], precision=_HIGH,
                preferred_element_type=jnp.float32)
        + jnp.dot(bb1[...], fc[D:2 * D], precision=_HIGH,
                  preferred_element_type=jnp.float32)
        + jnp.dot(bb2[...], fc[2 * D:3 * D], precision=_HIGH,
                  preferred_element_type=jnp.float32))


def _tc_fold(W0, W1, W2, FC1, FC2, b0, b1, b2):
    return pl.pallas_call(
        _fold_body,
        out_shape=[jax.ShapeDtypeStruct((D, 3 * F), jnp.float32),
                   jax.ShapeDtypeStruct((1, F), jnp.float32)],
    )(W0, W1, W2, FC1, FC2, b0, b1, b2)


BN = 1000


def _main_body(feat_b, m_b, d0_b, d1_b, y0, y1n, x2, nrm):
    nv = lax.rsqrt(jnp.maximum(d0_b[...] + d1_b[...], 1.0))
    y = jnp.dot(feat_b[...], m_b[...], precision=_HIGH,
                preferred_element_type=jnp.float32)
    y0[...] = y[:, 0:F]
    y1n[...] = y[:, F:2 * F] * nv
    x2[...] = y[:, 2 * F:3 * F] * nv
    nrm[...] = nv


def _tc_main(feat, M, d0, d1):
    bs = pl.BlockSpec((BN, F), lambda i: (i, 0))
    b1 = pl.BlockSpec((BN, 1), lambda i: (i, 0))
    return pl.pallas_call(
        _main_body,
        grid=(N // BN,),
        in_specs=[
            pl.BlockSpec((BN, D), lambda i: (i, 0)),
            pl.BlockSpec((D, 3 * F), lambda i: (0, 0)),
            b1,
            b1,
        ],
        out_specs=[bs, bs, bs, b1],
        out_shape=[jax.ShapeDtypeStruct((N, F), jnp.float32),
                   jax.ShapeDtypeStruct((N, F), jnp.float32),
                   jax.ShapeDtypeStruct((N, F), jnp.float32),
                   jax.ShapeDtypeStruct((N, 1), jnp.float32)],
    )(feat, M, d0, d1)


def _combine_body(y1n_b, nrm_b, r_b, xu):
    nv = nrm_b[...]
    xu[...] = y1n_b[...] + (nv * nv) * r_b[...]


def _tc_combine(Y1n, nrm, R):
    bs = pl.BlockSpec((BN, F), lambda i: (i, 0))
    return pl.pallas_call(
        _combine_body,
        grid=(N // BN,),
        in_specs=[bs, pl.BlockSpec((BN, 1), lambda i: (i, 0)), bs],
        out_specs=bs,
        out_shape=jax.ShapeDtypeStruct((N, F), jnp.float32),
    )(Y1n, nrm, R)


def _final_body(y0_b, nrm_b, q_b, c_b, out):
    out[...] = y0_b[...] + nrm_b[...] * q_b[...] + c_b[...]


def _tc_final(Y0, nrm, Q, cvec):
    bs = pl.BlockSpec((BN, F), lambda i: (i, 0))
    return pl.pallas_call(
        _final_body,
        grid=(N // BN,),
        in_specs=[bs, pl.BlockSpec((BN, 1), lambda i: (i, 0)), bs,
                  pl.BlockSpec((1, F), lambda i: (0, 0))],
        out_specs=bs,
        out_shape=jax.ShapeDtypeStruct((N, F), jnp.float32),
    )(Y0, nrm, Q, cvec)


def kernel(feat, edge_index, W0, b0, W1, b1, W2, b2, FC1, FC2):
    src = edge_index[0]
    dst = edge_index[1]
    srcp = src.reshape(ROWS, CW)
    dstp = dst.reshape(ROWS, CW)

    degP = _sc_degree()(dst).reshape(NC, NPAD)
    d0 = degP[0, :N].reshape(N, 1)
    d1 = degP[1, :N].reshape(N, 1)

    M, cvec = _tc_fold(W0, W1, W2, FC1, FC2,
                       b0.reshape(1, D), b1.reshape(1, D), b2.reshape(1, D))
    Y0, Y1n, X2, nrm = _tc_main(feat, M, d0, d1)

    R = _sc_prop()(X2, srcp, dstp)
    Xu = _tc_combine(Y1n, nrm, R[:N])
    Q = _sc_prop()(Xu, srcp, dstp)
    out = _tc_final(Y0, nrm, Q[:N], cvec)
    return out
